# baseline (speedup 1.0000x reference)
import jax
import jax.numpy as jnp
from jax.experimental import pallas as pl
from jax.experimental.pallas import tpu as pltpu

_E = 16
_IN = 1024
_OUT = 256
_B = 512
_NQ = 2
_CE = 2
_NCHUNK = _E // _CE


def _route(x, rw):
    logits = jnp.dot(x, rw, preferred_element_type=jnp.float32)
    m = jnp.max(logits, axis=1, keepdims=True)
    ex = jnp.exp(logits - m)
    probs = ex / jnp.sum(ex, axis=1, keepdims=True)
    iota = jax.lax.broadcasted_iota(jnp.int32, probs.shape, 1)
    p1 = jnp.max(probs, axis=1, keepdims=True)
    idx1 = jnp.min(jnp.where(probs == p1, iota, _E), axis=1, keepdims=True)
    m1 = (iota == idx1).astype(jnp.float32)
    probs2 = jnp.where(iota == idx1, -1.0, probs)
    p2 = jnp.max(probs2, axis=1, keepdims=True)
    idx2 = jnp.min(jnp.where(probs2 == p2, iota, _E), axis=1, keepdims=True)
    m2 = (iota == idx2).astype(jnp.float32)
    s = p1 + p2
    combine = (p1 / s) * m1 + (p2 / s) * m2
    importance = jnp.mean(probs, axis=0, keepdims=True)
    load = jnp.mean((combine > 0).astype(jnp.float32), axis=0, keepdims=True)
    aux = _E * jnp.sum(importance * load)
    return combine, aux


def _w_copy(ew_ref, wbuf_ref, sems, k):
    return pltpu.make_async_copy(
        ew_ref.at[pl.ds(k * _CE, _CE)],
        wbuf_ref.at[pl.ds(k * _CE, _CE)],
        sems.at[k % _NQ])


def _fused_kernel(x_ref, rw_ref, rb_ref, ew_ref, eb_ref,
                  out_ref, loss_ref, wbuf_ref, sems):

    x = x_ref[...]
    wc, wl = _route(x, rw_ref[...])
    bc, bl = _route(x, rb_ref[...])
    loss_ref[0, 0] = wl + bl
    xb = x.astype(jnp.bfloat16)
    acc = jnp.dot(bc, eb_ref[...], preferred_element_type=jnp.float32)

    iota = jax.lax.broadcasted_iota(jnp.int32, (_B, _E), 1)
    for k in range(_NCHUNK):
        for j in range(_CE):
            e = k * _CE + j
            y = jnp.dot(x, wbuf_ref[e],
                        preferred_element_type=jnp.float32)
            c_e = jnp.sum(jnp.where(iota == e, wc, 0.0), axis=1, keepdims=True)
            acc = acc + c_e * y
    out_ref[...] = acc


def kernel(input_batch, weight_router_w, bias_router_w, expert_weights, expert_biases):
    out, loss = pl.pallas_call(
        _fused_kernel,
        in_specs=[
            pl.BlockSpec(memory_space=pltpu.VMEM),
            pl.BlockSpec(memory_space=pltpu.VMEM),
            pl.BlockSpec(memory_space=pltpu.VMEM),
            pl.BlockSpec(memory_space=pl.ANY),
            pl.BlockSpec(memory_space=pltpu.VMEM),
        ],
        out_specs=[
            pl.BlockSpec(memory_space=pltpu.VMEM),
            pl.BlockSpec(memory_space=pltpu.SMEM),
        ],
        out_shape=[
            jax.ShapeDtypeStruct((_B, _OUT), jnp.float32),
            jax.ShapeDtypeStruct((1, 1), jnp.float32),
        ],
        scratch_shapes=[
            pltpu.VMEM((_E, _IN, _OUT), jnp.float32),
            pltpu.SemaphoreType.DMA((_NQ,)),
        ],
    )(input_batch, weight_router_w, bias_router_w, expert_weights, expert_biases)
    return out, loss[0, 0]


# speedup vs baseline: 1.0252x; 1.0252x over previous
import jax
import jax.numpy as jnp
from jax.experimental import pallas as pl
from jax.experimental.pallas import tpu as pltpu

_E = 16
_IN = 1024
_OUT = 256
_B = 512
_NQ = 4
_CE = _E // _NQ
_NEG = -3e38


def _routes_fused(x, rwb):
    logits = jnp.dot(x, rwb, preferred_element_type=jnp.float32)
    iota = jax.lax.broadcasted_iota(jnp.int32, logits.shape, 1)
    grp = iota >= _E

    def segmax(v, fill):
        a = jnp.max(jnp.where(grp, fill, v), axis=1, keepdims=True)
        b = jnp.max(jnp.where(grp, v, fill), axis=1, keepdims=True)
        return jnp.where(grp, b, a)

    def segsum(v):
        a = jnp.sum(jnp.where(grp, 0.0, v), axis=1, keepdims=True)
        b = jnp.sum(jnp.where(grp, v, 0.0), axis=1, keepdims=True)
        return jnp.where(grp, b, a)

    def segargmax(hit):
        a = jnp.min(jnp.where(hit & ~grp, iota, 2 * _E), axis=1, keepdims=True)
        b = jnp.min(jnp.where(hit & grp, iota, 2 * _E), axis=1, keepdims=True)
        return jnp.where(grp, b, a)

    ex = jnp.exp(logits - segmax(logits, _NEG))
    probs = ex / segsum(ex)
    p1 = segmax(probs, _NEG)
    idx1 = segargmax(probs == p1)
    m1 = iota == idx1
    probs2 = jnp.where(m1, -1.0, probs)
    p2 = segmax(probs2, _NEG)
    idx2 = segargmax(probs2 == p2)
    m2 = iota == idx2
    s = p1 + p2
    combine = jnp.where(m1, p1 / s, 0.0) + jnp.where(m2, p2 / s, 0.0)
    importance = jnp.mean(probs, axis=0, keepdims=True)
    load = jnp.mean((combine > 0).astype(jnp.float32), axis=0, keepdims=True)
    aux = _E * jnp.sum(importance * load)
    return combine, aux


def _w_copy(ew_ref, wbuf_ref, sems, k):
    return pltpu.make_async_copy(
        ew_ref.at[pl.ds(k * _CE, _CE)],
        wbuf_ref.at[pl.ds(k * _CE, _CE)],
        sems.at[k])


def _fused_kernel(x_ref, rwb_ref, ew_ref, eb_ref,
                  out_ref, loss_ref, wbuf_ref, sems):
    for k in range(_NQ):
        _w_copy(ew_ref, wbuf_ref, sems, k).start()

    x = x_ref[...]
    combine, aux = _routes_fused(x, rwb_ref[...])
    loss_ref[0, 0] = aux
    ebext = jnp.concatenate(
        [jnp.zeros((_E, _OUT), jnp.float32), eb_ref[...]], axis=0)
    acc = jnp.dot(combine, ebext, preferred_element_type=jnp.float32)

    iota = jax.lax.broadcasted_iota(jnp.int32, (_B, 2 * _E), 1)
    for k in range(_NQ):
        _w_copy(ew_ref, wbuf_ref, sems, k).wait()
        for j in range(_CE):
            e = k * _CE + j
            y = jnp.dot(x, wbuf_ref[e], preferred_element_type=jnp.float32)
            c_e = jnp.sum(jnp.where(iota == e, combine, 0.0), axis=1,
                          keepdims=True)
            acc = acc + c_e * y
    out_ref[...] = acc


def kernel(input_batch, weight_router_w, bias_router_w, expert_weights, expert_biases):
    rwb = jnp.concatenate([weight_router_w, bias_router_w], axis=1)
    out, loss = pl.pallas_call(
        _fused_kernel,
        in_specs=[
            pl.BlockSpec(memory_space=pltpu.VMEM),
            pl.BlockSpec(memory_space=pltpu.VMEM),
            pl.BlockSpec(memory_space=pl.ANY),
            pl.BlockSpec(memory_space=pltpu.VMEM),
        ],
        out_specs=[
            pl.BlockSpec(memory_space=pltpu.VMEM),
            pl.BlockSpec(memory_space=pltpu.SMEM),
        ],
        out_shape=[
            jax.ShapeDtypeStruct((_B, _OUT), jnp.float32),
            jax.ShapeDtypeStruct((1, 1), jnp.float32),
        ],
        scratch_shapes=[
            pltpu.VMEM((_E, _IN, _OUT), jnp.float32),
            pltpu.SemaphoreType.DMA((_NQ,)),
        ],
    )(input_batch, rwb, expert_weights, expert_biases)
    return out, loss[0, 0]
